# baseline (speedup 1.0000x reference)
import functools

import jax
import jax.numpy as jnp
from jax import lax
from jax.experimental import pallas as pl
from jax.experimental.pallas import tpu as pltpu
from jax.experimental.pallas import tpu_sc as plsc

NROWS = 1000000
HIDDEN = 32
BATCH = 4096

_INFO = plsc.get_sparse_core_info()
_NC = _INFO.num_cores
_NS = _INFO.num_subcores
_L = _INFO.num_lanes
_NW = _NC * _NS
_NB = 4
_NQ = HIDDEN // 8

_S_SC = 2560
_S_TC = BATCH - _S_SC
_BPW = _S_SC // _NW
_K = 8


def _fetch(tab, e, blk, b, sem):
    e128 = pl.multiple_of((e >> 7) << 7, 128)
    for q in range(_NQ):
        pltpu.async_copy(tab.at[q, :, pl.ds(e128, 128)], blk.at[b, q], sem[b])


def _drain(tab, blk, b, sem):
    pltpu.make_async_copy(tab.at[:, :, pl.ds(0, 128)], blk.at[b], sem[b]).wait()


def _sc_kernel(hidx_hbm, ridx_hbm, tidx_hbm, ent_hbm, rel_hbm, out_hbm,
               idx_h, idx_r, idx_t,
               blk_h, blk_r, blk_t, out_v,
               sem_h, sem_r, sem_t):
    wid = lax.axis_index("s") * _NC + lax.axis_index("c")
    base = wid * _BPW

    pltpu.sync_copy(hidx_hbm.at[pl.ds(base, _BPW)], idx_h.at[pl.ds(0, _BPW)])
    pltpu.sync_copy(ridx_hbm.at[pl.ds(base, _BPW)], idx_r.at[pl.ds(0, _BPW)])
    pltpu.sync_copy(tidx_hbm.at[pl.ds(base, _BPW)], idx_t.at[pl.ds(0, _BPW)])

    iota = lax.iota(jnp.int32, _L)

    def issue(eh, er, et, b):
        _fetch(ent_hbm, eh, blk_h, b, sem_h)
        _fetch(rel_hbm, er, blk_r, b, sem_r)
        _fetch(ent_hbm, et, blk_t, b, sem_t)

    ch0_h = idx_h[pl.ds(0, _L)]
    ch0_r = idx_r[pl.ds(0, _L)]
    ch0_t = idx_t[pl.ds(0, _L)]
    for b in range(_NB):
        issue(ch0_h[b], ch0_r[b], ch0_t[b], b)

    def sample_score(eh, er, et, b):
        gh = ((eh & 127) >> 4) << 4
        gr = ((er & 127) >> 4) << 4
        gt = ((et & 127) >> 4) << 4
        lh = jnp.full((_L,), eh & 15, jnp.int32)
        lr = jnp.full((_L,), er & 15, jnp.int32)
        lt = jnp.full((_L,), et & 15, jnp.int32)

        def chan(c, acc):
            q = c >> 3
            c8 = c & 7
            h = jnp.take(blk_h[b, q, c8, pl.ds(gh, _L)], lh)
            r = jnp.take(blk_r[b, q, c8, pl.ds(gr, _L)], lr)
            t = jnp.take(blk_t[b, q, c8, pl.ds(gt, _L)], lt)
            return acc + jnp.abs(h + r - t)

        return lax.fori_loop(0, HIDDEN, chan, jnp.zeros((_L,), jnp.float32))

    def chunk(k, carry):
        cur_h = idx_h[pl.ds(k * _L, _L)]
        cur_r = idx_r[pl.ds(k * _L, _L)]
        cur_t = idx_t[pl.ds(k * _L, _L)]
        nxt_h = idx_h[pl.ds(k * _L + _L, _L)]
        nxt_r = idx_r[pl.ds(k * _L + _L, _L)]
        nxt_t = idx_t[pl.ds(k * _L + _L, _L)]
        outacc = jnp.zeros((_L,), jnp.float32)
        for j in range(_L):
            b = j % _NB
            i = k * _L + j
            _drain(ent_hbm, blk_h, b, sem_h)
            _drain(rel_hbm, blk_r, b, sem_r)
            _drain(ent_hbm, blk_t, b, sem_t)
            s = sample_score(cur_h[j], cur_r[j], cur_t[j], b)
            outacc = jnp.where(iota == j, s, outacc)

            @pl.when(i + _NB < _BPW)
            def _():
                if j + _NB < _L:
                    issue(cur_h[j + _NB], cur_r[j + _NB], cur_t[j + _NB], b)
                else:
                    issue(nxt_h[j + _NB - _L], nxt_r[j + _NB - _L],
                          nxt_t[j + _NB - _L], b)

        out_v[pl.ds(k * _L, _L)] = outacc
        return carry

    lax.fori_loop(0, _BPW // _L, chunk, 0)

    pltpu.sync_copy(out_v, out_hbm.at[pl.ds(base, _BPW)])


@jax.jit
def _sc_scores(hidx, ridx, tidx, ent3, rel3):
    mesh = plsc.VectorSubcoreMesh(core_axis_name="c", subcore_axis_name="s")
    kern = functools.partial(
        pl.kernel,
        mesh=mesh,
        compiler_params=pltpu.CompilerParams(use_tc_tiling_on_sc=True),
        out_type=jax.ShapeDtypeStruct((_S_SC,), jnp.float32),
        scratch_types=[
            pltpu.VMEM((_BPW + _L,), jnp.int32),
            pltpu.VMEM((_BPW + _L,), jnp.int32),
            pltpu.VMEM((_BPW + _L,), jnp.int32),
            pltpu.VMEM((_NB, _NQ, 8, 128), jnp.float32),
            pltpu.VMEM((_NB, _NQ, 8, 128), jnp.float32),
            pltpu.VMEM((_NB, _NQ, 8, 128), jnp.float32),
            pltpu.VMEM((_BPW,), jnp.float32),
            [pltpu.SemaphoreType.DMA] * _NB,
            [pltpu.SemaphoreType.DMA] * _NB,
            [pltpu.SemaphoreType.DMA] * _NB,
        ],
    )(_sc_kernel)
    return kern(hidx, ridx, tidx, ent3, rel3)


def _tc_kernel(hp, rp, tp, *refs):
    out_ref = refs[-1]
    in_refs = refs[:-1]
    i = pl.program_id(0)
    lane = lax.broadcasted_iota(jnp.int32, (HIDDEN, 128), 1)
    scores = []
    for k in range(_K):
        eh = hp[i * _K + k] & 127
        er = rp[i * _K + k] & 127
        et = tp[i * _K + k] & 127
        bh = in_refs[k][...]
        br = in_refs[_K + k][...]
        bt = in_refs[2 * _K + k][...]
        merged = (jnp.where(lane == eh, bh, 0.0)
                  + jnp.where(lane == er, br, 0.0)
                  - jnp.where(lane == et, bt, 0.0))
        cols = jnp.sum(merged, axis=1)
        scores.append(jnp.sum(jnp.abs(cols)))
    out_ref[...] = jnp.stack(scores).reshape(1, 1, _K)


@jax.jit
def _tc_scores(hidx, ridx, tidx, ent_t, rel_t):
    def mk_spec(which):
        def im(i, hp, rp, tp, k):
            pref = (hp, rp, tp)[which]
            return (0, pref[i * _K + k] >> 7)
        return [pl.BlockSpec((HIDDEN, 128), functools.partial(im, k=k))
                for k in range(_K)]

    grid_spec = pltpu.PrefetchScalarGridSpec(
        num_scalar_prefetch=3,
        grid=(_S_TC // _K,),
        in_specs=mk_spec(0) + mk_spec(1) + mk_spec(2),
        out_specs=pl.BlockSpec((1, 1, _K), lambda i, hp, rp, tp: (i, 0, 0)),
    )
    out = pl.pallas_call(
        _tc_kernel,
        grid_spec=grid_spec,
        out_shape=jax.ShapeDtypeStruct((_S_TC // _K, 1, _K), jnp.float32),
    )(hidx, ridx, tidx, *([ent_t] * _K + [rel_t] * _K + [ent_t] * _K))
    return out.reshape(_S_TC)


def kernel(sample, entity_embedding, relation_embedding):
    ent_t = entity_embedding.T
    rel_t = relation_embedding.T
    ent3 = ent_t.reshape(_NQ, 8, NROWS)
    rel3 = rel_t.reshape(_NQ, 8, NROWS)
    h, r, t = sample[:, 0], sample[:, 1], sample[:, 2]
    sc_out = _sc_scores(h[:_S_SC], r[:_S_SC], t[:_S_SC], ent3, rel3)
    tc_out = _tc_scores(h[_S_SC:], r[_S_SC:], t[_S_SC:], ent_t, rel_t)
    return jnp.concatenate([sc_out, tc_out]).reshape(BATCH, 1)


# speedup vs baseline: 1.3526x; 1.3526x over previous
import functools

import jax
import jax.numpy as jnp
from jax import lax
from jax.experimental import pallas as pl
from jax.experimental.pallas import tpu as pltpu
from jax.experimental.pallas import tpu_sc as plsc

NROWS = 1000000
HIDDEN = 32
BATCH = 4096

_INFO = plsc.get_sparse_core_info()
_NC = _INFO.num_cores
_NS = _INFO.num_subcores
_L = _INFO.num_lanes
_NW = _NC * _NS
_NB = 4
_NQ = HIDDEN // 8

_S_SC = 3072
_S_TC = BATCH - _S_SC
_BPW = _S_SC // _NW
_K = 8


def _fetch(tab, e, blk, b, sem):
    e128 = pl.multiple_of((e >> 7) << 7, 128)
    for q in range(_NQ):
        pltpu.async_copy(tab.at[q, :, pl.ds(e128, 128)], blk.at[b, q], sem[b])


def _drain(tab, blk, b, sem):
    pltpu.make_async_copy(tab.at[:, :, pl.ds(0, 128)], blk.at[b], sem[b]).wait()


def _sc_kernel(hidx_hbm, ridx_hbm, tidx_hbm, ent_hbm, rel_hbm, out_hbm,
               idx_h, idx_r, idx_t,
               blk_h, blk_r, blk_t, out_v,
               sem_h, sem_r, sem_t):
    wid = lax.axis_index("s") * _NC + lax.axis_index("c")
    base = wid * _BPW

    pltpu.sync_copy(hidx_hbm.at[pl.ds(base, _BPW)], idx_h.at[pl.ds(0, _BPW)])
    pltpu.sync_copy(ridx_hbm.at[pl.ds(base, _BPW)], idx_r.at[pl.ds(0, _BPW)])
    pltpu.sync_copy(tidx_hbm.at[pl.ds(base, _BPW)], idx_t.at[pl.ds(0, _BPW)])

    iota = lax.iota(jnp.int32, _L)

    def issue(eh, er, et, b):
        _fetch(ent_hbm, eh, blk_h, b, sem_h)
        _fetch(rel_hbm, er, blk_r, b, sem_r)
        _fetch(ent_hbm, et, blk_t, b, sem_t)

    ch0_h = idx_h[pl.ds(0, _L)]
    ch0_r = idx_r[pl.ds(0, _L)]
    ch0_t = idx_t[pl.ds(0, _L)]
    for b in range(_NB):
        issue(ch0_h[b], ch0_r[b], ch0_t[b], b)

    def sample_score(eh, er, et, b):
        gh = ((eh & 127) >> 4) << 4
        gr = ((er & 127) >> 4) << 4
        gt = ((et & 127) >> 4) << 4
        lh = jnp.full((_L,), eh & 15, jnp.int32)
        lr = jnp.full((_L,), er & 15, jnp.int32)
        lt = jnp.full((_L,), et & 15, jnp.int32)

        def chan(c, acc):
            q = c >> 3
            c8 = c & 7
            h = jnp.take(blk_h[b, q, c8, pl.ds(gh, _L)], lh)
            r = jnp.take(blk_r[b, q, c8, pl.ds(gr, _L)], lr)
            t = jnp.take(blk_t[b, q, c8, pl.ds(gt, _L)], lt)
            return acc + jnp.abs(h + r - t)

        return lax.fori_loop(0, HIDDEN, chan, jnp.zeros((_L,), jnp.float32))

    def chunk(k, carry):
        cur_h = idx_h[pl.ds(k * _L, _L)]
        cur_r = idx_r[pl.ds(k * _L, _L)]
        cur_t = idx_t[pl.ds(k * _L, _L)]
        nxt_h = idx_h[pl.ds(k * _L + _L, _L)]
        nxt_r = idx_r[pl.ds(k * _L + _L, _L)]
        nxt_t = idx_t[pl.ds(k * _L + _L, _L)]
        outacc = jnp.zeros((_L,), jnp.float32)
        for j in range(_L):
            b = j % _NB
            i = k * _L + j
            _drain(ent_hbm, blk_h, b, sem_h)
            _drain(rel_hbm, blk_r, b, sem_r)
            _drain(ent_hbm, blk_t, b, sem_t)
            s = sample_score(cur_h[j], cur_r[j], cur_t[j], b)
            outacc = jnp.where(iota == j, s, outacc)

            @pl.when(i + _NB < _BPW)
            def _():
                if j + _NB < _L:
                    issue(cur_h[j + _NB], cur_r[j + _NB], cur_t[j + _NB], b)
                else:
                    issue(nxt_h[j + _NB - _L], nxt_r[j + _NB - _L],
                          nxt_t[j + _NB - _L], b)

        out_v[pl.ds(k * _L, _L)] = outacc
        return carry

    lax.fori_loop(0, _BPW // _L, chunk, 0)

    pltpu.sync_copy(out_v, out_hbm.at[pl.ds(base, _BPW)])


@jax.jit
def _sc_scores(hidx, ridx, tidx, ent3, rel3):
    mesh = plsc.VectorSubcoreMesh(core_axis_name="c", subcore_axis_name="s")
    kern = functools.partial(
        pl.kernel,
        mesh=mesh,
        compiler_params=pltpu.CompilerParams(use_tc_tiling_on_sc=True),
        cost_estimate=pl.CostEstimate(
            flops=3 * _S_SC * HIDDEN,
            transcendentals=0,
            bytes_accessed=3 * _S_SC * _NQ * 8 * 128 * 4,
        ),
        out_type=jax.ShapeDtypeStruct((_S_SC,), jnp.float32),
        scratch_types=[
            pltpu.VMEM((_BPW + _L,), jnp.int32),
            pltpu.VMEM((_BPW + _L,), jnp.int32),
            pltpu.VMEM((_BPW + _L,), jnp.int32),
            pltpu.VMEM((_NB, _NQ, 8, 128), jnp.float32),
            pltpu.VMEM((_NB, _NQ, 8, 128), jnp.float32),
            pltpu.VMEM((_NB, _NQ, 8, 128), jnp.float32),
            pltpu.VMEM((_BPW,), jnp.float32),
            [pltpu.SemaphoreType.DMA] * _NB,
            [pltpu.SemaphoreType.DMA] * _NB,
            [pltpu.SemaphoreType.DMA] * _NB,
        ],
    )(_sc_kernel)
    return kern(hidx, ridx, tidx, ent3, rel3)


def _tc_kernel(hp, rp, tp, *refs):
    out_ref = refs[-1]
    in_refs = refs[:-1]
    i = pl.program_id(0)
    lane = lax.broadcasted_iota(jnp.int32, (HIDDEN, 128), 1)
    scores = []
    for k in range(_K):
        eh = hp[i * _K + k] & 127
        er = rp[i * _K + k] & 127
        et = tp[i * _K + k] & 127
        bh = in_refs[k][...]
        br = in_refs[_K + k][...]
        bt = in_refs[2 * _K + k][...]
        merged = (jnp.where(lane == eh, bh, 0.0)
                  + jnp.where(lane == er, br, 0.0)
                  - jnp.where(lane == et, bt, 0.0))
        cols = jnp.sum(merged, axis=1)
        scores.append(jnp.sum(jnp.abs(cols)))
    out_ref[...] = jnp.stack(scores).reshape(1, 1, _K)


@jax.jit
def _tc_scores(hidx, ridx, tidx, ent_t, rel_t):
    def mk_spec(which):
        def im(i, hp, rp, tp, k):
            pref = (hp, rp, tp)[which]
            return (0, pref[i * _K + k] >> 7)
        return [pl.BlockSpec((HIDDEN, 128), functools.partial(im, k=k))
                for k in range(_K)]

    grid_spec = pltpu.PrefetchScalarGridSpec(
        num_scalar_prefetch=3,
        grid=(_S_TC // _K,),
        in_specs=mk_spec(0) + mk_spec(1) + mk_spec(2),
        out_specs=pl.BlockSpec((1, 1, _K), lambda i, hp, rp, tp: (i, 0, 0)),
    )
    out = pl.pallas_call(
        _tc_kernel,
        grid_spec=grid_spec,
        out_shape=jax.ShapeDtypeStruct((_S_TC // _K, 1, _K), jnp.float32),
    )(hidx, ridx, tidx, *([ent_t] * _K + [rel_t] * _K + [ent_t] * _K))
    return out.reshape(_S_TC)


def kernel(sample, entity_embedding, relation_embedding):
    ent_t = entity_embedding.T
    rel_t = relation_embedding.T
    ent3 = ent_t.reshape(_NQ, 8, NROWS)
    rel3 = rel_t.reshape(_NQ, 8, NROWS)
    h, r, t = sample[:, 0], sample[:, 1], sample[:, 2]
    sc_out = _sc_scores(h[:_S_SC], r[:_S_SC], t[:_S_SC], ent3, rel3)
    tc_out = _tc_scores(h[_S_SC:], r[_S_SC:], t[_S_SC:], ent_t, rel_t)
    return jnp.concatenate([sc_out, tc_out]).reshape(BATCH, 1)


# speedup vs baseline: 1.8283x; 1.3517x over previous
import functools

import jax
import jax.numpy as jnp
from jax import lax
from jax.experimental import pallas as pl
from jax.experimental.pallas import tpu as pltpu
from jax.experimental.pallas import tpu_sc as plsc

NROWS = 1000000
HIDDEN = 32
BATCH = 4096

_INFO = plsc.get_sparse_core_info()
_NC = _INFO.num_cores
_NS = _INFO.num_subcores
_L = _INFO.num_lanes
_NW = _NC * _NS
_BPW = BATCH // _NW
_NB = 4
_NQ = HIDDEN // 8


def _fetch(tab, e, blk, b, sem):
    e128 = pl.multiple_of((e >> 7) << 7, 128)
    for q in range(_NQ):
        pltpu.async_copy(tab.at[q, :, pl.ds(e128, 128)], blk.at[b, q], sem[b])


def _drain(tab, blk, b, sem):
    pltpu.make_async_copy(tab.at[:, :, pl.ds(0, 128)], blk.at[b], sem[b]).wait()


def _score_kernel(hidx_hbm, ridx_hbm, tidx_hbm, ent_hbm, rel_hbm, out_hbm,
                  idx_h, idx_r, idx_t,
                  blk_h, blk_r, blk_t, out_v,
                  sem_h, sem_r, sem_t):
    wid = lax.axis_index("s") * _NC + lax.axis_index("c")
    base = wid * _BPW

    pltpu.sync_copy(hidx_hbm.at[pl.ds(base, _BPW)], idx_h.at[pl.ds(0, _BPW)])
    pltpu.sync_copy(ridx_hbm.at[pl.ds(base, _BPW)], idx_r.at[pl.ds(0, _BPW)])
    pltpu.sync_copy(tidx_hbm.at[pl.ds(base, _BPW)], idx_t.at[pl.ds(0, _BPW)])

    iota = lax.iota(jnp.int32, _L)

    def issue(eh, er, et, b):
        _fetch(ent_hbm, eh, blk_h, b, sem_h)
        _fetch(rel_hbm, er, blk_r, b, sem_r)
        _fetch(ent_hbm, et, blk_t, b, sem_t)

    ch0_h = idx_h[pl.ds(0, _L)]
    ch0_r = idx_r[pl.ds(0, _L)]
    ch0_t = idx_t[pl.ds(0, _L)]
    for b in range(_NB):
        issue(ch0_h[b], ch0_r[b], ch0_t[b], b)

    def sample_score(eh, er, et, b):
        gh = ((eh & 127) >> 4) << 4
        gr = ((er & 127) >> 4) << 4
        gt = ((et & 127) >> 4) << 4
        lh = jnp.full((_L,), eh & 15, jnp.int32)
        lr = jnp.full((_L,), er & 15, jnp.int32)
        lt = jnp.full((_L,), et & 15, jnp.int32)

        def chan(c, acc):
            q = c >> 3
            c8 = c & 7
            h = jnp.take(blk_h[b, q, c8, pl.ds(gh, _L)], lh)
            r = jnp.take(blk_r[b, q, c8, pl.ds(gr, _L)], lr)
            t = jnp.take(blk_t[b, q, c8, pl.ds(gt, _L)], lt)
            return acc + jnp.abs(h + r - t)

        return lax.fori_loop(0, HIDDEN, chan, jnp.zeros((_L,), jnp.float32))

    def chunk(k, carry):
        cur_h = idx_h[pl.ds(k * _L, _L)]
        cur_r = idx_r[pl.ds(k * _L, _L)]
        cur_t = idx_t[pl.ds(k * _L, _L)]
        nxt_h = idx_h[pl.ds(k * _L + _L, _L)]
        nxt_r = idx_r[pl.ds(k * _L + _L, _L)]
        nxt_t = idx_t[pl.ds(k * _L + _L, _L)]
        outacc = jnp.zeros((_L,), jnp.float32)
        for j in range(_L):
            b = j % _NB
            i = k * _L + j
            _drain(ent_hbm, blk_h, b, sem_h)
            _drain(rel_hbm, blk_r, b, sem_r)
            _drain(ent_hbm, blk_t, b, sem_t)
            s = sample_score(cur_h[j], cur_r[j], cur_t[j], b)
            outacc = jnp.where(iota == j, s, outacc)

            @pl.when(i + _NB < _BPW)
            def _():
                if j + _NB < _L:
                    issue(cur_h[j + _NB], cur_r[j + _NB], cur_t[j + _NB], b)
                else:
                    issue(nxt_h[j + _NB - _L], nxt_r[j + _NB - _L],
                          nxt_t[j + _NB - _L], b)

        out_v[pl.ds(k * _L, _L)] = outacc
        return carry

    lax.fori_loop(0, _BPW // _L, chunk, 0)

    pltpu.sync_copy(out_v, out_hbm.at[pl.ds(base, _BPW)])


@jax.jit
def _scores(hidx, ridx, tidx, ent_t, rel_t):
    mesh = plsc.VectorSubcoreMesh(core_axis_name="c", subcore_axis_name="s")
    kern = functools.partial(
        pl.kernel,
        mesh=mesh,
        compiler_params=pltpu.CompilerParams(use_tc_tiling_on_sc=True),
        out_type=jax.ShapeDtypeStruct((BATCH,), jnp.float32),
        scratch_types=[
            pltpu.VMEM((_BPW + _L,), jnp.int32),
            pltpu.VMEM((_BPW + _L,), jnp.int32),
            pltpu.VMEM((_BPW + _L,), jnp.int32),
            pltpu.VMEM((_NB, _NQ, 8, 128), jnp.float32),
            pltpu.VMEM((_NB, _NQ, 8, 128), jnp.float32),
            pltpu.VMEM((_NB, _NQ, 8, 128), jnp.float32),
            pltpu.VMEM((_BPW,), jnp.float32),
            [pltpu.SemaphoreType.DMA] * _NB,
            [pltpu.SemaphoreType.DMA] * _NB,
            [pltpu.SemaphoreType.DMA] * _NB,
        ],
    )(_score_kernel)
    return kern(hidx, ridx, tidx, ent_t, rel_t)


def kernel(sample, entity_embedding, relation_embedding):
    ent3 = entity_embedding.T.reshape(_NQ, 8, NROWS)
    rel3 = relation_embedding.T.reshape(_NQ, 8, NROWS)
    out = _scores(sample[:, 0], sample[:, 1], sample[:, 2], ent3, rel3)
    return out.reshape(BATCH, 1)
